# baseline (speedup 1.0000x reference)
import functools

import jax
import jax.numpy as jnp
from jax import lax
from jax.experimental import pallas as pl
from jax.experimental.pallas import tpu as pltpu
from jax.experimental.pallas import tpu_sc as plsc

BATCH = 4096
SEQ_LEN = 200
EMBED_DIM = 64

NC = 2
NS = 16
NW = NC * NS
LANES = 16

BT = BATCH // 128
DT = EMBED_DIM // 8
NUNITS = SEQ_LEN
NBLOCKS = SEQ_LEN // 2
NBUF = 3


def _make_kernel():
  mesh = plsc.VectorSubcoreMesh(
      core_axis_name="c", subcore_axis_name="s",
      num_cores=NC, num_subcores=NS)

  @functools.partial(
      pl.kernel,
      out_type=jax.ShapeDtypeStruct((SEQ_LEN, DT, BT, 8, 128), jnp.float32),
      mesh=mesh,
      scratch_types=[
          pltpu.VMEM((NBUF, 2, 128), jnp.int32),
          pltpu.VMEM((NBUF, 2 * 128, EMBED_DIM), jnp.float32),
          pltpu.VMEM((128, 65), jnp.float32),
          pltpu.VMEM((NBUF, 2, DT, 8, 128), jnp.float32),
          pltpu.VMEM((SEQ_LEN, EMBED_DIM), jnp.float32),
          [pltpu.SemaphoreType.DMA] * NBUF,
          [pltpu.SemaphoreType.DMA] * NBUF,
          [pltpu.SemaphoreType.DMA] * NBUF,
      ],
      compiler_params=pltpu.CompilerParams(use_tc_tiling_on_sc=False,
                                           needs_layout_passes=False,
                                           disable_bounds_checks=True),
  )
  def emb_kernel(x5_hbm, table_hbm, pos_hbm, out_hbm,
                 idx_v, rows_v, rpad_v, obuf_v, pos_v, isem, gsem, wsem):
    cid = lax.axis_index("c")
    sid = lax.axis_index("s")
    wid = sid * NC + cid

    iota16 = lax.iota(jnp.int32, 16)
    row_idx = [iota16 + 16 * k for k in range(8)]

    def idx_start(blk, slot):
      u = blk * 2
      pltpu.async_copy(x5_hbm.at[u // 8, wid, pl.ds(u % 8, 2)],
                       idx_v.at[slot], isem[slot])

    def idx_wait(slot):
      pltpu.make_async_copy(x5_hbm.at[0, 0, pl.ds(0, 2)], idx_v.at[slot],
                            isem[slot]).wait()

    def gather_start(slot):
      for su in range(2):
        pltpu.async_copy(table_hbm.at[idx_v.at[slot, su]],
                         rows_v.at[slot, pl.ds(su * 128, 128)], gsem[slot])

    def gather_wait(slot):
      for su in range(2):
        pltpu.make_async_copy(table_hbm.at[idx_v.at[slot, su]],
                              rows_v.at[slot, pl.ds(su * 128, 128)],
                              gsem[slot]).wait()

    def wb_start(blk, slot):
      pltpu.async_copy(obuf_v.at[slot],
                       out_hbm.at[pl.ds(blk * 2, 2), :, wid], wsem[slot])

    def wb_wait(slot):
      pltpu.make_async_copy(obuf_v.at[slot],
                            out_hbm.at[pl.ds(0, 2), :, 0], wsem[slot]).wait()

    def transpose_block(blk, slot):
      for su in range(2):
        u = blk * 2 + su
        rows = rows_v.at[slot, pl.ds(su * 128, 128)]
        obuf = obuf_v.at[slot, su]

        prow = [pos_v[u, pl.ds(16 * q, 16)] for q in range(4)]

        @pl.loop(0, 128, unroll=2)
        def r_loop(r):
          for q in range(4):
            rpad_v[r, pl.ds(16 * q, 16)] = (rows[r, pl.ds(16 * q, 16)]
                                            + prow[q])

        @pl.loop(0, DT)
        def dt_loop(dt):
          obuf_dt = obuf.at[dt]
          d0 = dt * 8
          for dr in range(8):
            col_d = jnp.broadcast_to(d0 + dr, (16,))
            for k in range(8):
              obuf_dt[dr, pl.ds(16 * k, 16)] = plsc.load_gather(
                  rpad_v, [row_idx[k], col_d])

    pltpu.sync_copy(pos_hbm, pos_v)

    for c in range(3):
      idx_start(c, c)
    for c in range(2):
      idx_wait(c)
      gather_start(c)

    @pl.loop(0, NBLOCKS, step=NBUF)
    def main_loop(blk0):
      for b in range(NBUF):
        blk = blk0 + b

        @pl.when(jnp.logical_and(blk >= NBUF, blk < NBLOCKS))
        def _():
          wb_wait(b)

        @pl.when(blk + 2 < NBLOCKS)
        def _():
          idx_wait((b + 2) % NBUF)
          gather_start((b + 2) % NBUF)

        @pl.when(blk < NBLOCKS)
        def _():
          gather_wait(b)

        @pl.when(blk + 3 < NBLOCKS)
        def _():
          idx_start(blk + 3, b)

        @pl.when(blk < NBLOCKS)
        def _():
          transpose_block(blk, b)
          wb_start(blk, b)

    for b in range(NBUF):
      wb_wait(b)

  return emb_kernel


_emb_kernel = _make_kernel()


def kernel(x, word_table, pos_table):
  x5 = (x.astype(jnp.int32).T
        .reshape(SEQ_LEN // 8, 8, BT, 128)
        .transpose(0, 2, 1, 3))
  o5 = _emb_kernel(x5, word_table, pos_table[:SEQ_LEN])
  return o5.transpose(2, 4, 0, 1, 3).reshape(BATCH, SEQ_LEN, EMBED_DIM)


# speedup vs baseline: 1.3378x; 1.3378x over previous
import functools

import jax
import jax.numpy as jnp
from jax import lax
from jax.experimental import pallas as pl
from jax.experimental.pallas import tpu as pltpu
from jax.experimental.pallas import tpu_sc as plsc

BATCH = 4096
SEQ_LEN = 200
EMBED_DIM = 64
NUM_ROWS = BATCH * SEQ_LEN

NC = 2
NS = 16
NW = NC * NS

ROWS_PER_WORKER = NUM_ROWS // NW
SEQS_PER_CHUNK = 2
CHUNK = SEQS_PER_CHUNK * SEQ_LEN
NCHUNK = ROWS_PER_WORKER // CHUNK
NBUF = 4


def _make_kernel():
  mesh = plsc.VectorSubcoreMesh(
      core_axis_name="c", subcore_axis_name="s",
      num_cores=NC, num_subcores=NS)

  @functools.partial(
      pl.kernel,
      out_type=jax.ShapeDtypeStruct((NUM_ROWS, EMBED_DIM), jnp.float32),
      mesh=mesh,
      scratch_types=[
          pltpu.VMEM((NBUF, CHUNK), jnp.int32),
          pltpu.VMEM((NBUF, CHUNK, EMBED_DIM), jnp.float32),
          pltpu.VMEM_SHARED((CHUNK, EMBED_DIM), jnp.float32),
          [pltpu.SemaphoreType.DMA] * NBUF,
          [pltpu.SemaphoreType.DMA] * NBUF,
          [pltpu.SemaphoreType.DMA] * NBUF,
          [pltpu.SemaphoreType.DMA] * NBUF,
      ],
      compiler_params=pltpu.CompilerParams(use_tc_tiling_on_sc=False),
  )
  def emb_kernel(idx_hbm, table_hbm, pos_hbm, out_hbm,
                 idx_v, rows_v, pos_sh, isem, fsem, gsem, wsem):
    cid = lax.axis_index("c")
    sid = lax.axis_index("s")
    wid = sid * NC + cid
    wbase = wid * ROWS_PER_WORKER

    def idx_start(c, slot):
      pltpu.async_copy(idx_hbm.at[pl.ds(wbase + c * CHUNK, CHUNK)],
                       idx_v.at[slot], isem[slot])

    def idx_wait(slot):
      pltpu.make_async_copy(idx_hbm.at[pl.ds(0, CHUNK)],
                            idx_v.at[slot], isem[slot]).wait()

    def fill_start(slot):
      pltpu.async_copy(pos_sh, rows_v.at[slot], fsem[slot])

    def fill_wait(slot):
      pltpu.make_async_copy(pos_sh, rows_v.at[slot], fsem[slot]).wait()

    def gather_start(slot):
      pltpu.async_copy(table_hbm.at[idx_v.at[slot]], rows_v.at[slot],
                       gsem[slot], add=True)

    def gather_wait(slot):
      pltpu.make_async_copy(table_hbm.at[idx_v.at[slot]], rows_v.at[slot],
                            gsem[slot]).wait()

    def wb_start(c, slot):
      pltpu.async_copy(rows_v.at[slot],
                       out_hbm.at[pl.ds(wbase + c * CHUNK, CHUNK)],
                       wsem[slot])

    def wb_wait(slot):
      pltpu.make_async_copy(rows_v.at[slot],
                            out_hbm.at[pl.ds(0, CHUNK)], wsem[slot]).wait()

    def stage(i, b, do_wb_wait, do_idx, do_gather):
      if do_wb_wait:
        wb_wait((b + 2) % NBUF)
      if do_gather:
        fill_start((b + 2) % NBUF)
      if do_idx:
        idx_start(i + 3, (b + 3) % NBUF)
      if do_gather:
        idx_wait((b + 2) % NBUF)
        fill_wait((b + 2) % NBUF)
        gather_start((b + 2) % NBUF)
      gather_wait(b)
      wb_start(i, b)

    @pl.when(sid == 0)
    def _():
      for s in range(SEQS_PER_CHUNK):
        pltpu.sync_copy(pos_hbm.at[pl.ds(0, SEQ_LEN)],
                        pos_sh.at[pl.ds(s * SEQ_LEN, SEQ_LEN)])
    plsc.subcore_barrier()

    for c in range(3):
      idx_start(c, c)
    for c in range(2):
      fill_start(c)
    for c in range(2):
      idx_wait(c)
      fill_wait(c)
      gather_start(c)

    stage(0, 0, False, True, True)
    stage(1, 1, False, True, True)
    stage(2, 2, True, True, True)
    stage(3, 3, True, True, True)

    @pl.loop(4, NCHUNK - 4, step=NBUF)
    def main_loop(i):
      for b in range(NBUF):
        stage(i + b, b, True, True, True)

    stage(NCHUNK - 4, 0, True, True, True)
    stage(NCHUNK - 3, 1, True, False, True)
    stage(NCHUNK - 2, 2, True, False, False)
    stage(NCHUNK - 1, 3, True, False, False)
    wb_wait(2)
    wb_wait(3)

  return emb_kernel


_emb_kernel = _make_kernel()


def kernel(x, word_table, pos_table):
  xf = x.reshape(NUM_ROWS).astype(jnp.int32)
  out = _emb_kernel(xf, word_table, pos_table)
  return out.reshape(BATCH, SEQ_LEN, EMBED_DIM)
